# baseline (speedup 1.0000x reference)
import jax
import jax.numpy as jnp
from jax import lax
from jax.experimental import pallas as pl
from jax.experimental.pallas import tpu as pltpu
from jax.experimental.pallas import tpu_sc as plsc

BATCH = 1024
MAXLEN = 200
EMBED = 64
HALF = 100

NUM_CORES = 2
NUM_SUBCORES = 16
NUM_WORKERS = NUM_CORES * NUM_SUBCORES
SEQS_PER_WORKER = BATCH // NUM_WORKERS
PAIRS_PER_WORKER = SEQS_PER_WORKER // 2
LANES = 16
NBUF = 2
NROUNDS = PAIRS_PER_WORKER // NBUF


def _embed_kernel(x_hbm, tok_hbm, pos_hbm, out_hbm, idx_v, pos_v,
                  gbufs, pbufs, gsems, osems):
    wid = lax.axis_index("s") * NUM_CORES + lax.axis_index("c")
    sbase = wid * SEQS_PER_WORKER
    pbase = wid * PAIRS_PER_WORKER

    pltpu.sync_copy(x_hbm.at[pl.ds(sbase, SEQS_PER_WORKER)], idx_v)
    pltpu.sync_copy(pos_hbm, pos_v)

    def start_gathers(p, b):
        for e in (0, 1):
            pltpu.make_async_copy(
                tok_hbm.at[idx_v.at[2 * p + e, 0]],
                gbufs[b][e].at[pl.ds(0, HALF)], gsems[b]
            ).start()
            pltpu.make_async_copy(
                tok_hbm.at[idx_v.at[2 * p + e, 1]],
                gbufs[b][e].at[pl.ds(HALF, HALF)], gsems[b]
            ).start()

    def wait_gathers(p, b):
        for e in (0, 1):
            pltpu.make_async_copy(
                tok_hbm.at[idx_v.at[2 * p + e, 0]],
                gbufs[b][e].at[pl.ds(0, HALF)], gsems[b]
            ).wait()
            pltpu.make_async_copy(
                tok_hbm.at[idx_v.at[2 * p + e, 1]],
                gbufs[b][e].at[pl.ds(HALF, HALF)], gsems[b]
            ).wait()

    def add_pos(b):
        @pl.loop(0, MAXLEN)
        def _(r):
            for j in range(EMBED // LANES):
                c = pl.ds(j * LANES, LANES)
                cr = pl.ds(j * LANES + EMBED, LANES)
                pbufs[b][r, c] = gbufs[b][0][r, c] + pos_v[r, c]
                pbufs[b][r, cr] = gbufs[b][1][r, c] + pos_v[r, c]

    def start_write(p, b):
        pltpu.make_async_copy(
            pbufs[b], out_hbm.at[pl.ds((pbase + p) * MAXLEN, MAXLEN)], osems[b]
        ).start()

    def wait_write(p, b):
        pltpu.make_async_copy(
            pbufs[b], out_hbm.at[pl.ds((pbase + p) * MAXLEN, MAXLEN)], osems[b]
        ).wait()

    for b in range(NBUF):
        start_gathers(b, b)

    for b in range(NBUF):
        wait_gathers(b, b)
        add_pos(b)
        start_gathers(NBUF + b, b)
        start_write(b, b)

    @pl.loop(1, NROUNDS - 1)
    def _(g):
        for b in range(NBUF):
            p = g * NBUF + b
            wait_gathers(p, b)
            wait_write(p - NBUF, b)
            add_pos(b)
            start_gathers(p + NBUF, b)
            start_write(p, b)

    for b in range(NBUF):
        p = (NROUNDS - 1) * NBUF + b
        wait_gathers(p, b)
        wait_write(p - NBUF, b)
        add_pos(b)
        start_write(p, b)
    for b in range(NBUF):
        p = (NROUNDS - 1) * NBUF + b
        wait_write(p, b)


def _sc_gather_add(x3, token_table, pos_table):
    mesh = plsc.VectorSubcoreMesh(core_axis_name="c", subcore_axis_name="s")
    seqbuf = lambda: pltpu.VMEM((MAXLEN, EMBED), jnp.float32)

    def body(x_hbm, tok_hbm, pos_hbm, out_hbm, idx_v, pos_v,
             ga0, gb0, ga1, gb1, p0, p1, gs0, gs1, os0, os1):
        _embed_kernel(x_hbm, tok_hbm, pos_hbm, out_hbm, idx_v, pos_v,
                      ((ga0, gb0), (ga1, gb1)), (p0, p1),
                      (gs0, gs1), (os0, os1))

    k = pl.kernel(
        body,
        out_type=jax.ShapeDtypeStruct((BATCH // 2 * MAXLEN, 2 * EMBED),
                                      jnp.float32),
        mesh=mesh,
        scratch_types=[
            pltpu.VMEM((SEQS_PER_WORKER, 2, HALF), jnp.int32),
            seqbuf(), seqbuf(), seqbuf(), seqbuf(), seqbuf(),
            pltpu.VMEM((MAXLEN, 2 * EMBED), jnp.float32),
            pltpu.VMEM((MAXLEN, 2 * EMBED), jnp.float32),
            pltpu.SemaphoreType.DMA,
            pltpu.SemaphoreType.DMA,
            pltpu.SemaphoreType.DMA,
            pltpu.SemaphoreType.DMA,
        ],
        compiler_params=pltpu.CompilerParams(use_tc_tiling_on_sc=False),
    )
    return k(x3, token_table, pos_table)


PAIR_BLK = 4


def _split_body(pair_ref, out_ref):
    for k in range(PAIR_BLK):
        t = pair_ref[pl.ds(k * MAXLEN, MAXLEN), :]
        out_ref[2 * k] = t[:, :EMBED]
        out_ref[2 * k + 1] = t[:, EMBED:]


def _tc_split(pair_flat):
    return pl.pallas_call(
        _split_body,
        out_shape=jax.ShapeDtypeStruct((BATCH, MAXLEN, EMBED), jnp.float32),
        grid=(BATCH // (2 * PAIR_BLK),),
        in_specs=[
            pl.BlockSpec((PAIR_BLK * MAXLEN, 2 * EMBED), lambda i: (i, 0)),
        ],
        out_specs=pl.BlockSpec((2 * PAIR_BLK, MAXLEN, EMBED),
                               lambda i: (i, 0, 0)),
    )(pair_flat)


@jax.jit
def kernel(x, token_table, pos_table):
    x3 = x.reshape(BATCH, 2, HALF).astype(jnp.int32)
    pair_flat = _sc_gather_add(x3, token_table, pos_table)
    return _tc_split(pair_flat)


# speedup vs baseline: 2.3884x; 2.3884x over previous
import jax
import jax.numpy as jnp
from jax import lax
from jax.experimental import pallas as pl
from jax.experimental.pallas import tpu as pltpu
from jax.experimental.pallas import tpu_sc as plsc

BATCH = 1024
MAXLEN = 200
EMBED = 64
HALF = 100
H_PAD = 104

NUM_CORES = 2
NUM_SUBCORES = 16
NUM_WORKERS = NUM_CORES * NUM_SUBCORES
SEQS_PER_WORKER = BATCH // NUM_WORKERS
LANES = 16
NBUF = 2
NROUNDS = SEQS_PER_WORKER // NBUF


def _embed_kernel(x_hbm, tok_hbm, pos_hbm, out_hbm, idx_v, pos_v,
                  gbufs, wbufs, gsems, osems):
    wid = lax.axis_index("s") * NUM_CORES + lax.axis_index("c")
    base = wid * SEQS_PER_WORKER

    pltpu.sync_copy(x_hbm.at[pl.ds(base, SEQS_PER_WORKER)], idx_v)
    pltpu.sync_copy(pos_hbm, pos_v)

    def start_gather(s, b):
        pltpu.make_async_copy(
            tok_hbm.at[idx_v.at[s, 0]], gbufs[b].at[pl.ds(0, HALF)], gsems[b]
        ).start()
        pltpu.make_async_copy(
            tok_hbm.at[idx_v.at[s, 1]], gbufs[b].at[pl.ds(HALF, HALF)], gsems[b]
        ).start()

    def wait_gather(s, b):
        pltpu.make_async_copy(
            tok_hbm.at[idx_v.at[s, 0]], gbufs[b].at[pl.ds(0, HALF)], gsems[b]
        ).wait()
        pltpu.make_async_copy(
            tok_hbm.at[idx_v.at[s, 1]], gbufs[b].at[pl.ds(HALF, HALF)], gsems[b]
        ).wait()

    def add_pos(b):
        @pl.loop(0, HALF)
        def _(h):
            r = 2 * h
            for j in range(EMBED // LANES):
                c = pl.ds(j * LANES, LANES)
                cl = pl.ds(j * LANES + EMBED, LANES)
                wbufs[b][h, c] = gbufs[b][r, c] + pos_v[r, c]
                wbufs[b][h, cl] = gbufs[b][r + 1, c] + pos_v[r + 1, c]

    def start_write(s, b):
        pltpu.make_async_copy(
            wbufs[b], out_hbm.at[base + s, pl.ds(0, HALF)], osems[b]
        ).start()

    def wait_write(s, b):
        pltpu.make_async_copy(
            wbufs[b], out_hbm.at[base + s, pl.ds(0, HALF)], osems[b]
        ).wait()

    for b in range(NBUF):
        start_gather(b, b)

    for b in range(NBUF):
        wait_gather(b, b)
        add_pos(b)
        start_gather(NBUF + b, b)
        start_write(b, b)

    @pl.loop(1, NROUNDS - 1)
    def _(g):
        for b in range(NBUF):
            s = g * NBUF + b
            wait_gather(s, b)
            wait_write(s - NBUF, b)
            add_pos(b)
            start_gather(s + NBUF, b)
            start_write(s, b)

    for b in range(NBUF):
        s = (NROUNDS - 1) * NBUF + b
        wait_gather(s, b)
        wait_write(s - NBUF, b)
        add_pos(b)
        start_write(s, b)
    for b in range(NBUF):
        s = (NROUNDS - 1) * NBUF + b
        wait_write(s, b)


def _wrapped(x3, token_table, pos_table):
    mesh = plsc.VectorSubcoreMesh(core_axis_name="c", subcore_axis_name="s")
    vmem_rows = lambda: pltpu.VMEM((MAXLEN, EMBED), jnp.float32)

    def body(x_hbm, tok_hbm, pos_hbm, out_hbm, idx_v, pos_v,
             g0, g1, w0, w1, gs0, gs1, os0, os1):
        _embed_kernel(x_hbm, tok_hbm, pos_hbm, out_hbm, idx_v, pos_v,
                      (g0, g1), (w0, w1), (gs0, gs1), (os0, os1))

    k = pl.kernel(
        body,
        out_type=jax.ShapeDtypeStruct((BATCH, H_PAD, 2 * EMBED), jnp.float32),
        mesh=mesh,
        scratch_types=[
            pltpu.VMEM((SEQS_PER_WORKER, 2, HALF), jnp.int32),
            vmem_rows(), vmem_rows(), vmem_rows(),
            pltpu.VMEM((HALF, 2 * EMBED), jnp.float32),
            pltpu.VMEM((HALF, 2 * EMBED), jnp.float32),
            pltpu.SemaphoreType.DMA,
            pltpu.SemaphoreType.DMA,
            pltpu.SemaphoreType.DMA,
            pltpu.SemaphoreType.DMA,
        ],
        compiler_params=pltpu.CompilerParams(use_tc_tiling_on_sc=False),
    )
    return k(x3, token_table, pos_table)


B_BLK = 128
H_BLK = H_PAD


def _xpose_body(j_ref, out_ref):
    for hh in range(HALF):
        out_ref[pl.ds(hh * 2 * EMBED, 2 * EMBED), :] = j_ref[:, hh, :].T


def _tc_xpose(jflat):
    return pl.pallas_call(
        _xpose_body,
        out_shape=jax.ShapeDtypeStruct((HALF * 2 * EMBED, BATCH), jnp.float32),
        grid=(BATCH // B_BLK,),
        in_specs=[
            pl.BlockSpec((B_BLK, H_BLK, 2 * EMBED), lambda i: (i, 0, 0)),
        ],
        out_specs=pl.BlockSpec((H_BLK * 2 * EMBED, B_BLK), lambda i: (0, i)),
    )(jflat)


@jax.jit
def kernel(x, token_table, pos_table):
    x3 = x.reshape(BATCH, 2, HALF).astype(jnp.int32)
    j = _wrapped(x3, token_table, pos_table)
    out_t = _tc_xpose(j)
    return jnp.transpose(out_t.reshape(MAXLEN, EMBED, BATCH), (2, 0, 1))
